# Initial kernel scaffold; baseline (speedup 1.0000x reference)
import jax
import jax.numpy as jnp
from jax.experimental import pallas as pl


def kernel(x, data):
    raise NotImplementedError("write your pallas kernel here")


# speedup vs baseline: 264.6181x; 264.6181x over previous
import functools

import jax
import jax.numpy as jnp
import numpy as np
from jax import lax
from jax._src import config as _jax_config
from jax.experimental import pallas as pl
from jax.experimental.pallas import tpu as pltpu
from jax.experimental.pallas import tpu_sc as plsc

_NL = 16
_F = 2
_N_PTS = 262144
_PRIME = 524309
_M = 19349663
_SCALES = [16 * 2 ** i for i in range(_NL)]
_OFFS = [0, 289, 1378, 5603, 22244, 88293, 351462, 875771, 1400080,
         1924389, 2448698, 2973007, 3497316, 4021625, 4545934, 5070243]
_SH = 6
_C_MODP = _M % _PRIME
_D_MODP = (_M * 1024) % _PRIME
_INVP = float(np.float32(1.0) / np.float32(_PRIME))

_NW = 32
_MB = 128
_PTS_W = _N_PTS // _NW
_NMB = _PTS_W // _MB
_LV = 4
_VROWS = _LV * 4
_NROW = (_NL - _LV) * 4
_TAB = _OFFS[_LV]


def _hash_y(yi):
    vlow = yi * _M
    b = vlow & 0xFFFFF
    yp = jnp.where(yi >= _PRIME, yi - _PRIME, yi)
    u = (yp >> 10) * _D_MODP + (yp & 1023) * _C_MODP
    q = (u.astype(jnp.float32) * _INVP).astype(jnp.int32)
    r = u - q * _PRIME
    r = jnp.where(r < 0, r + _PRIME, r)
    r = jnp.where(r >= _PRIME, r - _PRIME, r)
    return r, b


def _hash_mix(r, b, xi):
    s = r + ((b ^ xi) - b) + 2 * _PRIME
    q = (s.astype(jnp.float32) * _INVP).astype(jnp.int32)
    h = s - q * _PRIME
    h = jnp.where(h < 0, h + _PRIME, h)
    h = jnp.where(h >= _PRIME, h - _PRIME, h)
    return h


def _corners(px, py, l):
    sf = float(_SCALES[l])
    fx = px * sf
    fy = py * sf
    xi0 = fx.astype(jnp.int32)
    yi0 = fy.astype(jnp.int32)
    xi1 = (fx + 1.0).astype(jnp.int32)
    yi1 = (fy + 1.0).astype(jnp.int32)
    if l < _SH:
        sp1 = _SCALES[l] + 1
        r0 = xi0 * sp1 + _OFFS[l]
        r1 = xi1 * sp1 + _OFFS[l]
        return r0 + yi0, r0 + yi1, r1 + yi0, r1 + yi1
    ra, ba = _hash_y(yi0)
    rb, bb = _hash_y(yi1)
    return (_hash_mix(ra, ba, xi0) + _OFFS[l],
            _hash_mix(rb, bb, xi0) + _OFFS[l],
            _hash_mix(ra, ba, xi1) + _OFFS[l],
            _hash_mix(rb, bb, xi1) + _OFFS[l])


def _weights(px, py, l):
    sf = float(_SCALES[l])
    fx = px * sf
    fy = py * sf
    xf = fx - fx.astype(jnp.int32).astype(jnp.float32)
    yf = fy - fy.astype(jnp.int32).astype(jnp.float32)
    wx0 = 1.0 - xf
    wy0 = 1.0 - yf
    return wx0 * wy0, wx0 * yf, xf * wy0, xf * yf


_mesh = plsc.VectorSubcoreMesh(core_axis_name="c", subcore_axis_name="s")


@functools.partial(
    pl.kernel,
    out_type=jax.ShapeDtypeStruct((_NL * _F, _N_PTS), jnp.float32),
    mesh=_mesh,
    compiler_params=pltpu.CompilerParams(needs_layout_passes=False),
    scratch_types=[
        pltpu.VMEM((_PTS_W,), jnp.float32),
        pltpu.VMEM((_PTS_W,), jnp.float32),
        pltpu.VMEM((_TAB,), jnp.float32),
        pltpu.VMEM((_TAB,), jnp.float32),
        pltpu.VMEM((_NROW * _MB,), jnp.int32),
        pltpu.VMEM((_NROW * _MB,), jnp.int32),
        pltpu.VMEM((_NROW * _MB,), jnp.float32),
        pltpu.VMEM((_NROW * _MB,), jnp.float32),
        pltpu.VMEM((_NROW * _MB,), jnp.float32),
        pltpu.VMEM((_NROW * _MB,), jnp.float32),
        pltpu.VMEM((_NL * _F, _MB), jnp.float32),
        pltpu.SemaphoreType.DMA,
        pltpu.SemaphoreType.DMA,
    ],
)
def _plane_sc(px_hbm, py_hbm, d0_hbm, d1_hbm, out_hbm,
              pxb, pyb, t0b, t1b, idx0, idx1, a0b, a1b, b0b, b1b,
              outb, sem0, sem1):
    wid = lax.axis_index("s") * np.int32(2) + lax.axis_index("c")
    base0 = wid * np.int32(_PTS_W)
    pltpu.sync_copy(px_hbm.at[pl.ds(base0, _PTS_W)], pxb)
    pltpu.sync_copy(py_hbm.at[pl.ds(base0, _PTS_W)], pyb)
    pltpu.sync_copy(d0_hbm.at[pl.ds(0, _TAB)], t0b)
    pltpu.sync_copy(d1_hbm.at[pl.ds(0, _TAB)], t1b)

    def build(obase, idxb):
        @pl.loop(np.int32(0), np.int32(_MB // 16))
        def _(j):
            jo = j * np.int32(16)
            px = pxb[pl.ds(obase + jo, 16)]
            py = pyb[pl.ds(obase + jo, 16)]
            for l in range(_LV, _NL):
                i00, i01, i10, i11 = _corners(px, py, l)
                base = 4 * (l - _LV) * _MB
                idxb[pl.ds(np.int32(base + 0 * _MB) + jo, 16)] = i00
                idxb[pl.ds(np.int32(base + 1 * _MB) + jo, 16)] = i01
                idxb[pl.ds(np.int32(base + 2 * _MB) + jo, 16)] = i10
                idxb[pl.ds(np.int32(base + 3 * _MB) + jo, 16)] = i11

    def fire(idxb, v0b, v1b, sem):
        pltpu.make_async_copy(d0_hbm.at[idxb], v0b, sem).start()
        pltpu.make_async_copy(d1_hbm.at[idxb], v1b, sem).start()

    def drain(idxb, v0b, v1b, sem):
        pltpu.make_async_copy(d0_hbm.at[idxb], v0b, sem).wait()
        pltpu.make_async_copy(d1_hbm.at[idxb], v1b, sem).wait()

    def combine(obase, v0b, v1b):
        @pl.loop(np.int32(0), np.int32(_MB // 16))
        def _(j):
            jo = j * np.int32(16)
            px = pxb[pl.ds(obase + jo, 16)]
            py = pyb[pl.ds(obase + jo, 16)]
            for l in range(_LV):
                i00, i01, i10, i11 = _corners(px, py, l)
                ws = _weights(px, py, l)
                acc0 = ws[0] * plsc.load_gather(t0b, [i00])
                acc1 = ws[0] * plsc.load_gather(t1b, [i00])
                for c, ind in ((1, i01), (2, i10), (3, i11)):
                    acc0 = acc0 + ws[c] * plsc.load_gather(t0b, [ind])
                    acc1 = acc1 + ws[c] * plsc.load_gather(t1b, [ind])
                outb[2 * l, pl.ds(jo, 16)] = acc0
                outb[2 * l + 1, pl.ds(jo, 16)] = acc1
            for l in range(_LV, _NL):
                ws = _weights(px, py, l)
                base = 4 * (l - _LV) * _MB
                acc0 = ws[0] * v0b[pl.ds(np.int32(base) + jo, 16)]
                acc1 = ws[0] * v1b[pl.ds(np.int32(base) + jo, 16)]
                for c in range(1, 4):
                    o_c = np.int32(base + c * _MB)
                    acc0 = acc0 + ws[c] * v0b[pl.ds(o_c + jo, 16)]
                    acc1 = acc1 + ws[c] * v1b[pl.ds(o_c + jo, 16)]
                outb[2 * l, pl.ds(jo, 16)] = acc0
                outb[2 * l + 1, pl.ds(jo, 16)] = acc1

    build(np.int32(0), idx0)
    fire(idx0, a0b, a1b, sem0)

    @pl.loop(np.int32(0), np.int32(_NMB), step=np.int32(2))
    def mb_pair(mb):
        ob0 = mb * np.int32(_MB)
        ob1 = ob0 + np.int32(_MB)
        build(ob1, idx1)
        fire(idx1, b0b, b1b, sem1)
        drain(idx0, a0b, a1b, sem0)
        combine(ob0, a0b, a1b)
        pltpu.sync_copy(outb, out_hbm.at[:, pl.ds(base0 + ob0, _MB)])

        @pl.when(mb + np.int32(2) < np.int32(_NMB))
        def _():
            build(ob1 + np.int32(_MB), idx0)
            fire(idx0, a0b, a1b, sem0)

        drain(idx1, b0b, b1b, sem1)
        combine(ob1, b0b, b1b)
        pltpu.sync_copy(outb, out_hbm.at[:, pl.ds(base0 + ob1, _MB)])


def kernel(x, data):
    with _jax_config.enable_x64(False):
        px = x[:, 0]
        py = x[:, 1]
        d0 = data[:, 0]
        d1 = data[:, 1]
        out_t = _plane_sc(px, py, d0, d1)
        return out_t.T
